# baseline (speedup 1.0000x reference)
import functools

import jax
import jax.numpy as jnp
import numpy as np
from jax import lax
from jax.experimental import pallas as pl
from jax.experimental.pallas import tpu as pltpu
from jax.experimental.pallas import tpu_sc as plsc

N = 10000
E = 320000
R = 16
D = 128
H = 128
EPS = 1e-10

NC = 2
NS = 16
NW = NC * NS
CH = 80
EPW = E // NW
NPAD = 10240
RPT = NPAD // NS
NCH_M = EPW // CH
NR_T = N * R // NS

_ROWS = 400

_mesh = plsc.VectorSubcoreMesh(core_axis_name="c", subcore_axis_name="s")


def _memset16(ref, nwords, value=0.0):
    def body(i, carry):
        ref[pl.ds(i * 16, 16)] = jnp.full((16,), value, jnp.float32)
        return carry
    lax.fori_loop(0, nwords // 16, body, 0)


@functools.partial(
    pl.kernel,
    mesh=_mesh,
    out_type=jax.ShapeDtypeStruct((NC * N * R,), jnp.float32),
    scratch_types=[
        pltpu.VMEM_SHARED((N * R,), jnp.float32),
        pltpu.VMEM((2, CH), jnp.int32),
        pltpu.VMEM((2, CH), jnp.int32),
        pltpu.VMEM((2, CH), jnp.int32),
        pltpu.VMEM((CH,), jnp.float32),
        pltpu.VMEM((2000,), jnp.float32),
        pltpu.SemaphoreType.DMA,
        pltpu.SemaphoreType.DMA,
        pltpu.SemaphoreType.DMA,
        pltpu.SemaphoreType.DMA,
    ],
)
def _sc_deg(dst_hbm, type_hbm, deg_hbm, deg, ddv, dtv, didx, ones, zbuf,
            sl0, sl1, ss0, ss1):
    c = lax.axis_index("c")
    s = lax.axis_index("s")
    sem_ld = (sl0, sl1)
    sem_sc = (ss0, ss1)

    _memset16(zbuf, 2000)
    _memset16(ones, CH, 1.0)
    for z in range(NR_T // 2000):
        pltpu.sync_copy(zbuf, deg.at[pl.ds(s * NR_T + z * 2000, 2000)])
    plsc.subcore_barrier()

    def d_issue_loads(ci, b):
        off = c * (E // NC) + s * EPW + ci * CH
        pltpu.async_copy(dst_hbm.at[pl.ds(off, CH)], ddv.at[b], sem_ld[b])
        pltpu.async_copy(type_hbm.at[pl.ds(off, CH)], dtv.at[b], sem_ld[b])

    def d_wait_loads(b):
        pltpu.make_async_copy(
            dst_hbm.at[pl.ds(0, CH)], ddv.at[b], sem_ld[b]).wait()
        pltpu.make_async_copy(
            type_hbm.at[pl.ds(0, CH)], dtv.at[b], sem_ld[b]).wait()

    def d_wait_scatter(b):
        pltpu.make_async_copy(ones, deg.at[didx.at[b]], sem_sc[b]).wait()

    d_issue_loads(0, 0)

    def d_step(ci, b):
        nb = 1 - b
        d_wait_loads(b)

        @pl.when(ci + 1 <= NCH_M - 1)
        def _():
            d_issue_loads(ci + 1, nb)

        @pl.when(ci >= 2)
        def _():
            d_wait_scatter(b)

        for u in range(CH // 16):
            slo = pl.ds(u * 16, 16)
            didx[b, slo] = ddv[b, slo] * R + dtv[b, slo]
        pltpu.async_copy(ones, deg.at[didx.at[b]], sem_sc[b], add=True)

    def d_body(g, carry):
        d_step(2 * g, 0)
        d_step(2 * g + 1, 1)
        return carry

    lax.fori_loop(0, (NCH_M - 1) // 2, d_body, 0)
    d_step(NCH_M - 1, 0)
    d_wait_scatter(0)
    d_wait_scatter(1)
    plsc.subcore_barrier()
    def out_block(z, carry):
        off = s * NR_T + z * 2000
        pltpu.sync_copy(deg.at[pl.ds(off, 2000)], zbuf)
        pltpu.sync_copy(zbuf, deg_hbm.at[pl.ds(c * (N * R) + off, 2000)])
        return carry

    lax.fori_loop(0, NR_T // 2000, out_block, 0)


@functools.partial(
    pl.kernel,
    mesh=_mesh,
    out_type=jax.ShapeDtypeStruct((NC, NPAD, H), jnp.float32),
    scratch_types=[
        pltpu.VMEM_SHARED((NPAD, H), jnp.float32),
        pltpu.VMEM((4, CH), jnp.int32),
        pltpu.VMEM((4, CH), jnp.int32),
        pltpu.VMEM((4, CH), jnp.int32),
        pltpu.VMEM((2, CH), jnp.int32),
        pltpu.VMEM((2, CH), jnp.int32),
        pltpu.VMEM((2, CH), jnp.float32),
        pltpu.VMEM((2, CH, H), jnp.float32),
        pltpu.VMEM((2, CH, H), jnp.float32),
        pltpu.VMEM((32, H), jnp.float32),
        pltpu.SemaphoreType.DMA,
        pltpu.SemaphoreType.DMA,
        pltpu.SemaphoreType.DMA,
        pltpu.SemaphoreType.DMA,
        pltpu.SemaphoreType.DMA,
        pltpu.SemaphoreType.DMA,
        pltpu.SemaphoreType.DMA,
        pltpu.SemaphoreType.DMA,
        pltpu.SemaphoreType.DMA,
        pltpu.SemaphoreType.DMA,
    ],
)
def _sc_agg(src_hbm, type_hbm, dst_hbm, winv_hbm, y_hbm, out_hbm,
            agg, sv, tv, dv, giv, didx, wv, rows, frows, zrows,
            sl0, sl1, sl2, sl3, sg0, sg1, sw0, sw1, ss0, ss1):
    c = lax.axis_index("c")
    s = lax.axis_index("s")
    wid = s * NC + c
    sem_ld = (sl0, sl1, sl2, sl3)
    sem_g = (sg0, sg1)
    sem_w = (sw0, sw1)
    sem_sc = (ss0, ss1)

    for zr in range(32):
        for u in range(H // 16):
            zrows[zr, pl.ds(u * 16, 16)] = jnp.zeros((16,), jnp.float32)
    for z in range(RPT // 32):
        pltpu.sync_copy(zrows, agg.at[pl.ds(s * RPT + z * 32, 32)])
    plsc.subcore_barrier()

    def issue_loads(ci, r):
        base = wid * EPW + ci * CH
        pltpu.async_copy(src_hbm.at[pl.ds(base, CH)], sv.at[r], sem_ld[r])
        pltpu.async_copy(type_hbm.at[pl.ds(base, CH)], tv.at[r], sem_ld[r])
        pltpu.async_copy(dst_hbm.at[pl.ds(base, CH)], dv.at[r], sem_ld[r])

    def wait_loads(r):
        pltpu.make_async_copy(src_hbm.at[pl.ds(0, CH)], sv.at[r], sem_ld[r]).wait()
        pltpu.make_async_copy(type_hbm.at[pl.ds(0, CH)], tv.at[r], sem_ld[r]).wait()
        pltpu.make_async_copy(dst_hbm.at[pl.ds(0, CH)], dv.at[r], sem_ld[r]).wait()

    def prep_and_gather(b, r):
        for u in range(CH // 16):
            sl = pl.ds(u * 16, 16)
            giv[b, sl] = sv[r, sl] * R + tv[r, sl]
            didx[b, sl] = dv[r, sl] * R + tv[r, sl]
        pltpu.async_copy(y_hbm.at[giv.at[b]], rows.at[b], sem_g[b])
        pltpu.async_copy(winv_hbm.at[didx.at[b]], wv.at[b], sem_w[b])

    def wait_gathers(b):
        pltpu.make_async_copy(y_hbm.at[giv.at[b]], rows.at[b], sem_g[b]).wait()
        pltpu.make_async_copy(winv_hbm.at[didx.at[b]], wv.at[b], sem_w[b]).wait()

    def wait_scatter(b, r):
        pltpu.make_async_copy(frows.at[b], agg.at[dv.at[r]], sem_sc[b]).wait()

    def scale_and_scatter(b, r):
        for g in range(CH // 16):
            wvec = wv[b, pl.ds(g * 16, 16)]
            for l in range(16):
                i = g * 16 + l
                wsc = wvec[l]
                for u in range(H // 16):
                    sl = pl.ds(u * 16, 16)
                    frows[b, i, sl] = rows[b, i, sl] * wsc
        pltpu.async_copy(frows.at[b], agg.at[dv.at[r]], sem_sc[b], add=True)

    issue_loads(0, 0)
    issue_loads(1, 1)
    wait_loads(0)
    prep_and_gather(0, 0)

    def step(ci, k):
        b = k % 2
        nb = 1 - b
        r = k % 4
        wait_gathers(b)

        @pl.when(ci >= 1)
        def _():
            wait_scatter(nb, (k + 3) % 4)

        @pl.when(ci + 2 <= NCH_M - 1)
        def _():
            issue_loads(ci + 2, (k + 2) % 4)

        @pl.when(ci + 1 <= NCH_M - 1)
        def _():
            wait_loads((k + 1) % 4)
            prep_and_gather(nb, (k + 1) % 4)

        scale_and_scatter(b, r)

    def body(g, carry):
        for k in range(4):
            step(4 * g + k, k)
        return carry

    lax.fori_loop(0, NCH_M // 4, body, 0)
    for k in range(4 * (NCH_M // 4), NCH_M):
        step(k, k % 4)
    wait_scatter((NCH_M - 1) % 2, (NCH_M - 1) % 4)
    plsc.subcore_barrier()
    pltpu.sync_copy(agg.at[pl.ds(s * RPT, RPT)], out_hbm.at[c, pl.ds(s * RPT, RPT)])


def _winv_body(d0_ref, d1_ref, o_ref):
    o_ref[...] = 1.0 / (d0_ref[...] + d1_ref[...] + EPS)


def _tc_winv(degp):
    d0 = degp[:N * R].reshape((N * R) // 128, 128)
    d1 = degp[N * R:].reshape((N * R) // 128, 128)
    return pl.pallas_call(
        _winv_body,
        out_shape=jax.ShapeDtypeStruct(((N * R) // 128, 128), jnp.float32),
    )(d0, d1).reshape(N * R)


def _mm_body(x_ref, w_ref, o_ref):
    o_ref[...] = jnp.dot(
        x_ref[...], w_ref[...], preferred_element_type=jnp.float32)


def _matmul_bf16(x, w):
    n, d = x.shape
    _, m = w.shape
    return pl.pallas_call(
        _mm_body,
        grid=(n // _ROWS,),
        in_specs=[
            pl.BlockSpec((_ROWS, d), lambda i: (i, 0)),
            pl.BlockSpec((d, m), lambda i: (0, 0)),
        ],
        out_specs=pl.BlockSpec((_ROWS, m), lambda i: (i, 0)),
        out_shape=jax.ShapeDtypeStruct((n, m), jnp.float32),
    )(x, w)


def _comb_mid_body(agg_ref, x_ref, ws_ref, b_ref, w2_ref, h_ref, y_ref):
    h = agg_ref[0] + agg_ref[1] + jnp.dot(
        x_ref[...], ws_ref[...], preferred_element_type=jnp.float32) + b_ref[...]
    h = jnp.maximum(h, 0.0)
    h_ref[...] = h
    y_ref[...] = jnp.dot(
        h.astype(jnp.bfloat16), w2_ref[...],
        preferred_element_type=jnp.float32)


def _combine_mid(aggp, x, ws, bvec, w2next):
    n = x.shape[0]
    m = w2next.shape[1]
    return pl.pallas_call(
        _comb_mid_body,
        grid=(n // _ROWS,),
        in_specs=[
            pl.BlockSpec((2, _ROWS, H), lambda i: (0, i, 0)),
            pl.BlockSpec((_ROWS, D), lambda i: (i, 0)),
            pl.BlockSpec((D, H), lambda i: (0, 0)),
            pl.BlockSpec((1, H), lambda i: (0, 0)),
            pl.BlockSpec((H, m), lambda i: (0, 0)),
        ],
        out_specs=[
            pl.BlockSpec((_ROWS, H), lambda i: (i, 0)),
            pl.BlockSpec((_ROWS, m), lambda i: (i, 0)),
        ],
        out_shape=[
            jax.ShapeDtypeStruct((n, H), jnp.float32),
            jax.ShapeDtypeStruct((n, m), jnp.float32),
        ],
    )(aggp, x, ws, bvec, w2next)


def _comb_last_body(agg_ref, x_ref, ws_ref, b_ref, h_ref, g_ref):
    i = pl.program_id(0)
    h = agg_ref[0] + agg_ref[1] + jnp.dot(
        x_ref[...], ws_ref[...], preferred_element_type=jnp.float32) + b_ref[...]
    h = jnp.maximum(h, 0.0)
    h_ref[...] = h

    @pl.when(i == 0)
    def _():
        g_ref[...] = jnp.zeros_like(g_ref)

    g_ref[...] += jnp.sum(h, axis=0, keepdims=True)


def _combine_last(aggp, x, ws, bvec):
    n = x.shape[0]
    return pl.pallas_call(
        _comb_last_body,
        grid=(n // _ROWS,),
        in_specs=[
            pl.BlockSpec((2, _ROWS, H), lambda i: (0, i, 0)),
            pl.BlockSpec((_ROWS, D), lambda i: (i, 0)),
            pl.BlockSpec((D, H), lambda i: (0, 0)),
            pl.BlockSpec((1, H), lambda i: (0, 0)),
        ],
        out_specs=[
            pl.BlockSpec((_ROWS, H), lambda i: (i, 0)),
            pl.BlockSpec((1, H), lambda i: (0, 0)),
        ],
        out_shape=[
            jax.ShapeDtypeStruct((n, H), jnp.float32),
            jax.ShapeDtypeStruct((1, H), jnp.float32),
        ],
    )(aggp, x, ws, bvec)


def kernel(x, edge_index, edge_type, Wl1, bl1, Ws1, bs1, Wl2, bl2, Ws2, bs2):
    src = edge_index[0]
    dst = edge_index[1]

    winv = _tc_winv(_sc_deg(dst, edge_type))

    W2_1 = Wl1.reshape(R, D, H).transpose(1, 0, 2).reshape(D, R * H)
    W2_2 = Wl2.reshape(R, H, H).transpose(1, 0, 2).reshape(H, R * H)
    W2_1p = W2_1.astype(jnp.bfloat16)
    W2_2p = W2_2.astype(jnp.bfloat16)

    Y1 = _matmul_bf16(x.astype(jnp.bfloat16), W2_1p)
    aggp1 = _sc_agg(src, edge_type, dst, winv, Y1.reshape(N * R, H))
    h1, Y2 = _combine_mid(aggp1, x, Ws1, (bl1 + bs1).reshape(1, H), W2_2p)
    aggp2 = _sc_agg(src, edge_type, dst, winv, Y2.reshape(N * R, H))
    h2, gsum = _combine_last(aggp2, h1, Ws2, (bl2 + bs2).reshape(1, H))
    return (gsum, h2)


# speedup vs baseline: 1.0082x; 1.0082x over previous
import functools

import jax
import jax.numpy as jnp
import numpy as np
from jax import lax
from jax.experimental import pallas as pl
from jax.experimental.pallas import tpu as pltpu
from jax.experimental.pallas import tpu_sc as plsc

N = 10000
E = 320000
R = 16
D = 128
H = 128
EPS = 1e-10

NC = 2
NS = 16
NW = NC * NS
CH = 80
CHD = 160
EPW = E // NW
EPT_DEG = E // NS
NPAD = 10240
RPT = NPAD // NS
NCH_M = EPW // CH
NCH_D = EPT_DEG // CHD
NR_T = N * R // NS

_ROWS = 400

_mesh = plsc.VectorSubcoreMesh(core_axis_name="c", subcore_axis_name="s")


def _memset16(ref, nwords, value=0.0):
    def body(i, carry):
        ref[pl.ds(i * 16, 16)] = jnp.full((16,), value, jnp.float32)
        return carry
    lax.fori_loop(0, nwords // 16, body, 0)


@functools.partial(
    pl.kernel,
    mesh=_mesh,
    out_type=jax.ShapeDtypeStruct((N * R,), jnp.float32),
    scratch_types=[
        pltpu.VMEM_SHARED((N * R,), jnp.float32),
        pltpu.VMEM((2, 2, CH), jnp.int32),
        pltpu.VMEM((2, 2, CH), jnp.int32),
        pltpu.VMEM((2, 2, CH), jnp.int32),
        pltpu.VMEM((CH,), jnp.float32),
        pltpu.VMEM((2000,), jnp.float32),
        pltpu.SemaphoreType.DMA,
        pltpu.SemaphoreType.DMA,
        pltpu.SemaphoreType.DMA,
        pltpu.SemaphoreType.DMA,
    ],
)
def _sc_winv(dst_hbm, type_hbm, winv_hbm, deg, ddv, dtv, didx, ones, zbuf,
             sl0, sl1, ss0, ss1):
    c = lax.axis_index("c")
    s = lax.axis_index("s")
    sem_ld = (sl0, sl1)
    sem_sc = (ss0, ss1)

    _memset16(zbuf, 2000)
    _memset16(ones, CH, 1.0)
    for z in range(NR_T // 2000):
        pltpu.sync_copy(zbuf, deg.at[pl.ds(s * NR_T + z * 2000, 2000)])
    plsc.subcore_barrier()

    def d_issue_loads(ci, b):
        base = s * EPT_DEG + ci * CHD
        for half in range(2):
            off = base + half * CH
            pltpu.async_copy(dst_hbm.at[pl.ds(off, CH)], ddv.at[b, half], sem_ld[b])
            pltpu.async_copy(type_hbm.at[pl.ds(off, CH)], dtv.at[b, half], sem_ld[b])

    def d_wait_loads(b):
        for half in range(2):
            pltpu.make_async_copy(
                dst_hbm.at[pl.ds(0, CH)], ddv.at[b, half], sem_ld[b]).wait()
            pltpu.make_async_copy(
                type_hbm.at[pl.ds(0, CH)], dtv.at[b, half], sem_ld[b]).wait()

    def d_wait_scatter(b):
        pltpu.make_async_copy(ones, deg.at[didx.at[b, 0]], sem_sc[b]).wait()
        pltpu.make_async_copy(ones, deg.at[didx.at[b, 1]], sem_sc[b]).wait()

    d_issue_loads(0, 0)

    def d_step(ci, b):
        nb = 1 - b
        d_wait_loads(b)

        @pl.when(ci + 1 <= NCH_D - 1)
        def _():
            d_issue_loads(ci + 1, nb)

        @pl.when(ci >= 2)
        def _():
            d_wait_scatter(b)

        for half in range(2):
            for u in range(CH // 16):
                slo = pl.ds(u * 16, 16)
                didx[b, half, slo] = ddv[b, half, slo] * R + dtv[b, half, slo]
        pltpu.async_copy(ones, deg.at[didx.at[b, 0]], sem_sc[b], add=True)
        pltpu.async_copy(ones, deg.at[didx.at[b, 1]], sem_sc[b], add=True)

    def d_body(g, carry):
        d_step(2 * g, 0)
        d_step(2 * g + 1, 1)
        return carry

    lax.fori_loop(0, NCH_D // 2, d_body, 0)
    d_step(NCH_D - 1, 0)
    d_wait_scatter(0)
    d_wait_scatter(1)
    plsc.subcore_barrier()

    @pl.when(c == 0)
    def _():
        def inv_block(z, carry):
            off = s * NR_T + z * 2000
            pltpu.sync_copy(deg.at[pl.ds(off, 2000)], zbuf)

            def inv16(i, carry2):
                sl = pl.ds(i * 16, 16)
                zbuf[sl] = 1.0 / (zbuf[sl] + EPS)
                return carry2

            lax.fori_loop(0, 2000 // 16, inv16, 0)
            pltpu.sync_copy(zbuf, winv_hbm.at[pl.ds(off, 2000)])
            return carry

        lax.fori_loop(0, NR_T // 2000, inv_block, 0)


@functools.partial(
    pl.kernel,
    mesh=_mesh,
    out_type=jax.ShapeDtypeStruct((NC, NPAD, H), jnp.float32),
    scratch_types=[
        pltpu.VMEM_SHARED((NPAD, H), jnp.float32),
        pltpu.VMEM((4, CH), jnp.int32),
        pltpu.VMEM((4, CH), jnp.int32),
        pltpu.VMEM((4, CH), jnp.int32),
        pltpu.VMEM((2, CH), jnp.int32),
        pltpu.VMEM((2, CH), jnp.int32),
        pltpu.VMEM((2, CH), jnp.float32),
        pltpu.VMEM((2, CH, H), jnp.float32),
        pltpu.VMEM((2, CH, H), jnp.float32),
        pltpu.VMEM((32, H), jnp.float32),
        pltpu.SemaphoreType.DMA,
        pltpu.SemaphoreType.DMA,
        pltpu.SemaphoreType.DMA,
        pltpu.SemaphoreType.DMA,
        pltpu.SemaphoreType.DMA,
        pltpu.SemaphoreType.DMA,
        pltpu.SemaphoreType.DMA,
        pltpu.SemaphoreType.DMA,
        pltpu.SemaphoreType.DMA,
        pltpu.SemaphoreType.DMA,
    ],
)
def _sc_agg(src_hbm, type_hbm, dst_hbm, winv_hbm, y_hbm, out_hbm,
            agg, sv, tv, dv, giv, didx, wv, rows, frows, zrows,
            sl0, sl1, sl2, sl3, sg0, sg1, sw0, sw1, ss0, ss1):
    c = lax.axis_index("c")
    s = lax.axis_index("s")
    wid = s * NC + c
    sem_ld = (sl0, sl1, sl2, sl3)
    sem_g = (sg0, sg1)
    sem_w = (sw0, sw1)
    sem_sc = (ss0, ss1)

    for zr in range(32):
        for u in range(H // 16):
            zrows[zr, pl.ds(u * 16, 16)] = jnp.zeros((16,), jnp.float32)
    for z in range(RPT // 32):
        pltpu.sync_copy(zrows, agg.at[pl.ds(s * RPT + z * 32, 32)])
    plsc.subcore_barrier()

    def issue_loads(ci, r):
        base = wid * EPW + ci * CH
        pltpu.async_copy(src_hbm.at[pl.ds(base, CH)], sv.at[r], sem_ld[r])
        pltpu.async_copy(type_hbm.at[pl.ds(base, CH)], tv.at[r], sem_ld[r])
        pltpu.async_copy(dst_hbm.at[pl.ds(base, CH)], dv.at[r], sem_ld[r])

    def wait_loads(r):
        pltpu.make_async_copy(src_hbm.at[pl.ds(0, CH)], sv.at[r], sem_ld[r]).wait()
        pltpu.make_async_copy(type_hbm.at[pl.ds(0, CH)], tv.at[r], sem_ld[r]).wait()
        pltpu.make_async_copy(dst_hbm.at[pl.ds(0, CH)], dv.at[r], sem_ld[r]).wait()

    def prep_and_gather(b, r):
        for u in range(CH // 16):
            sl = pl.ds(u * 16, 16)
            giv[b, sl] = sv[r, sl] * R + tv[r, sl]
            didx[b, sl] = dv[r, sl] * R + tv[r, sl]
        pltpu.async_copy(y_hbm.at[giv.at[b]], rows.at[b], sem_g[b])
        pltpu.async_copy(winv_hbm.at[didx.at[b]], wv.at[b], sem_w[b])

    def wait_gathers(b):
        pltpu.make_async_copy(y_hbm.at[giv.at[b]], rows.at[b], sem_g[b]).wait()
        pltpu.make_async_copy(winv_hbm.at[didx.at[b]], wv.at[b], sem_w[b]).wait()

    def wait_scatter(b, r):
        pltpu.make_async_copy(frows.at[b], agg.at[dv.at[r]], sem_sc[b]).wait()

    def scale_and_scatter(b, r):
        for g in range(CH // 16):
            wvec = wv[b, pl.ds(g * 16, 16)]
            for l in range(16):
                i = g * 16 + l
                wsc = wvec[l]
                for u in range(H // 16):
                    sl = pl.ds(u * 16, 16)
                    frows[b, i, sl] = rows[b, i, sl] * wsc
        pltpu.async_copy(frows.at[b], agg.at[dv.at[r]], sem_sc[b], add=True)

    issue_loads(0, 0)
    issue_loads(1, 1)
    wait_loads(0)
    prep_and_gather(0, 0)

    def step(ci, k):
        b = k % 2
        nb = 1 - b
        r = k % 4
        wait_gathers(b)

        @pl.when(ci >= 1)
        def _():
            wait_scatter(nb, (k + 3) % 4)

        @pl.when(ci + 2 <= NCH_M - 1)
        def _():
            issue_loads(ci + 2, (k + 2) % 4)

        @pl.when(ci + 1 <= NCH_M - 1)
        def _():
            wait_loads((k + 1) % 4)
            prep_and_gather(nb, (k + 1) % 4)

        scale_and_scatter(b, r)

    def body(g, carry):
        for k in range(4):
            step(4 * g + k, k)
        return carry

    lax.fori_loop(0, NCH_M // 4, body, 0)
    for k in range(4 * (NCH_M // 4), NCH_M):
        step(k, k % 4)
    wait_scatter((NCH_M - 1) % 2, (NCH_M - 1) % 4)
    plsc.subcore_barrier()
    pltpu.sync_copy(agg.at[pl.ds(s * RPT, RPT)], out_hbm.at[c, pl.ds(s * RPT, RPT)])


def _mm_body(x_ref, w_ref, o_ref):
    o_ref[...] = jnp.dot(
        x_ref[...], w_ref[...], preferred_element_type=jnp.float32)


def _matmul_bf16(x, w):
    n, d = x.shape
    _, m = w.shape
    return pl.pallas_call(
        _mm_body,
        grid=(n // _ROWS,),
        in_specs=[
            pl.BlockSpec((_ROWS, d), lambda i: (i, 0)),
            pl.BlockSpec((d, m), lambda i: (0, 0)),
        ],
        out_specs=pl.BlockSpec((_ROWS, m), lambda i: (i, 0)),
        out_shape=jax.ShapeDtypeStruct((n, m), jnp.float32),
    )(x, w)


def _comb_mid_body(agg_ref, x_ref, ws_ref, b_ref, w2_ref, h_ref, y_ref):
    h = agg_ref[0] + agg_ref[1] + jnp.dot(
        x_ref[...], ws_ref[...], preferred_element_type=jnp.float32) + b_ref[...]
    h = jnp.maximum(h, 0.0)
    h_ref[...] = h
    y_ref[...] = jnp.dot(
        h.astype(jnp.bfloat16), w2_ref[...],
        preferred_element_type=jnp.float32)


def _combine_mid(aggp, x, ws, bvec, w2next):
    n = x.shape[0]
    m = w2next.shape[1]
    return pl.pallas_call(
        _comb_mid_body,
        grid=(n // _ROWS,),
        in_specs=[
            pl.BlockSpec((2, _ROWS, H), lambda i: (0, i, 0)),
            pl.BlockSpec((_ROWS, D), lambda i: (i, 0)),
            pl.BlockSpec((D, H), lambda i: (0, 0)),
            pl.BlockSpec((1, H), lambda i: (0, 0)),
            pl.BlockSpec((H, m), lambda i: (0, 0)),
        ],
        out_specs=[
            pl.BlockSpec((_ROWS, H), lambda i: (i, 0)),
            pl.BlockSpec((_ROWS, m), lambda i: (i, 0)),
        ],
        out_shape=[
            jax.ShapeDtypeStruct((n, H), jnp.float32),
            jax.ShapeDtypeStruct((n, m), jnp.float32),
        ],
    )(aggp, x, ws, bvec, w2next)


def _comb_last_body(agg_ref, x_ref, ws_ref, b_ref, h_ref, g_ref):
    i = pl.program_id(0)
    h = agg_ref[0] + agg_ref[1] + jnp.dot(
        x_ref[...], ws_ref[...], preferred_element_type=jnp.float32) + b_ref[...]
    h = jnp.maximum(h, 0.0)
    h_ref[...] = h

    @pl.when(i == 0)
    def _():
        g_ref[...] = jnp.zeros_like(g_ref)

    g_ref[...] += jnp.sum(h, axis=0, keepdims=True)


def _combine_last(aggp, x, ws, bvec):
    n = x.shape[0]
    return pl.pallas_call(
        _comb_last_body,
        grid=(n // _ROWS,),
        in_specs=[
            pl.BlockSpec((2, _ROWS, H), lambda i: (0, i, 0)),
            pl.BlockSpec((_ROWS, D), lambda i: (i, 0)),
            pl.BlockSpec((D, H), lambda i: (0, 0)),
            pl.BlockSpec((1, H), lambda i: (0, 0)),
        ],
        out_specs=[
            pl.BlockSpec((_ROWS, H), lambda i: (i, 0)),
            pl.BlockSpec((1, H), lambda i: (0, 0)),
        ],
        out_shape=[
            jax.ShapeDtypeStruct((n, H), jnp.float32),
            jax.ShapeDtypeStruct((1, H), jnp.float32),
        ],
    )(aggp, x, ws, bvec)


def kernel(x, edge_index, edge_type, Wl1, bl1, Ws1, bs1, Wl2, bl2, Ws2, bs2):
    src = edge_index[0]
    dst = edge_index[1]

    winv = _sc_winv(dst, edge_type)

    W2_1 = Wl1.reshape(R, D, H).transpose(1, 0, 2).reshape(D, R * H)
    W2_2 = Wl2.reshape(R, H, H).transpose(1, 0, 2).reshape(H, R * H)
    W2_1p = W2_1.astype(jnp.bfloat16)
    W2_2p = W2_2.astype(jnp.bfloat16)

    Y1 = _matmul_bf16(x.astype(jnp.bfloat16), W2_1p)
    aggp1 = _sc_agg(src, edge_type, dst, winv, Y1.reshape(N * R, H))
    h1, Y2 = _combine_mid(aggp1, x, Ws1, (bl1 + bs1).reshape(1, H), W2_2p)
    aggp2 = _sc_agg(src, edge_type, dst, winv, Y2.reshape(N * R, H))
    h2, gsum = _combine_last(aggp2, h1, Ws2, (bl2 + bs2).reshape(1, H))
    return (gsum, h2)
